# baseline (speedup 1.0000x reference)
import jax
import jax.numpy as jnp
from jax.experimental import pallas as pl
from jax.experimental.pallas import tpu as pltpu

_BM = 1024
_NSPLIT = 2


def _encoder_body(*refs):
    x_refs = refs[:_NSPLIT]
    w1_ref, b1_ref, w2_ref, b2_ref = refs[_NSPLIT:_NSPLIT + 4]
    o_refs = refs[_NSPLIT + 4:]
    w1 = w1_ref[...].astype(jnp.bfloat16)
    w2 = w2_ref[...].astype(jnp.bfloat16)
    for x_ref, o_ref in zip(x_refs, o_refs):
        h = jax.lax.dot_general(
            x_ref[...].astype(jnp.bfloat16), w1,
            dimension_numbers=(((1,), (0,)), ((), ())),
            preferred_element_type=jnp.float32,
        )
        h = jnp.maximum(h + b1_ref[...], 0.0)
        z = jax.lax.dot_general(
            h.astype(jnp.bfloat16), w2,
            dimension_numbers=(((1,), (0,)), ((), ())),
            preferred_element_type=jnp.float32,
        )
        o_ref[...] = z + b2_ref[...]


def kernel(inputs, enc_w1, enc_b1, enc_w2, enc_b2,
           dec_w1, dec_b1, dec_w2, dec_b2, prior):
    del dec_w1, dec_b1, dec_w2, dec_b2, prior
    b, feat = inputs.shape
    hid = enc_w1.shape[1]
    code = enc_w2.shape[1]
    rows_per_stream = b // _NSPLIT
    steps = rows_per_stream // _BM
    blocks_per_stream = rows_per_stream // _BM

    x_specs = [
        pl.BlockSpec((_BM, feat), lambda i, s=s: (s * blocks_per_stream + i, 0))
        for s in range(_NSPLIT)
    ]
    w_specs = [
        pl.BlockSpec((feat, hid), lambda i: (0, 0)),
        pl.BlockSpec((1, hid), lambda i: (0, 0)),
        pl.BlockSpec((hid, code), lambda i: (0, 0)),
        pl.BlockSpec((1, code), lambda i: (0, 0)),
    ]
    out_specs = [pl.BlockSpec((_BM, code), lambda i: (i, 0))
                 for _ in range(_NSPLIT)]
    outs = pl.pallas_call(
        _encoder_body,
        grid=(steps,),
        in_specs=x_specs + w_specs,
        out_specs=out_specs,
        out_shape=[jax.ShapeDtypeStruct((rows_per_stream, code), jnp.float32)
                   for _ in range(_NSPLIT)],
        compiler_params=pltpu.CompilerParams(
            dimension_semantics=("arbitrary",),
        ),
    )(*([inputs] * _NSPLIT), enc_w1, enc_b1.reshape(1, hid),
      enc_w2, enc_b2.reshape(1, code))
    return jnp.concatenate(outs, axis=0)
